# baseline (device time: 150706 ns/iter reference)
import jax
import jax.numpy as jnp
from jax import lax
from jax.experimental import pallas as pl
from jax.experimental.pallas import tpu as pltpu

N_Z = 4


def kernel(O, Wo):
    B, S, H, D = O.shape
    K = H * D
    N = Wo.shape[1]
    S_out = S // N_Z
    HALF = S_out // 2
    NQ = N // 4

    o = O.reshape(B, S, K)
    q_idx = 2 * lax.axis_index("x") + lax.axis_index("y")
    w = lax.dynamic_slice(Wo, (0, q_idx * NQ), (K, NQ)).astype(jnp.bfloat16)

    def body(o_ref, w_ref, out_ref, send_ref, recv_ref, pbuf_ref,
             send_sems, recv_sems, xy_send_sems, xy_recv_sems, credit_sem):
        mx = lax.axis_index("x")
        my = lax.axis_index("y")
        mz = lax.axis_index("z")
        left = lax.rem(mz + N_Z - 1, N_Z)
        right = lax.rem(mz + 1, N_Z)

        qv = 2 * mx + my
        qx = 2 * (1 - mx) + my
        qy = 2 * mx + (1 - my)

        c0 = lax.rem(mz + 3, N_Z)
        c1 = lax.rem(mz + 2, N_Z)
        c2 = lax.rem(mz + 1, N_Z)
        c3 = mz

        barrier = pltpu.get_barrier_semaphore()
        for dev in ((mx, my, left), (mx, my, right),
                    (1 - mx, my, mz), (mx, 1 - my, mz)):
            pl.semaphore_signal(
                barrier, inc=1, device_id=dev,
                device_id_type=pl.DeviceIdType.MESH,
            )
        pl.semaphore_wait(barrier, 4)

        def phalf(h):
            return pbuf_ref.at[:, pl.ds(h * HALF, HALF), :]

        def ocol(qq):
            return out_ref.at[:, :, pl.ds(qq * NQ, NQ)]

        def ocol_half(qq, h):
            return out_ref.at[:, pl.ds(h * HALF, HALF), pl.ds(qq * NQ, NQ)]

        def partial_half(dst_ref, c, h):
            row0 = c * S_out + h * HALF
            for b in range(B):
                o_blk = o_ref[b, pl.ds(row0, HALF), :].astype(jnp.bfloat16)
                p = jnp.dot(
                    o_blk, w_ref[:, :], preferred_element_type=jnp.float32,
                )
                dst_ref[b, :, :] = p.astype(jnp.bfloat16)

        def add_half(dst_ref, a_ref, r_ref):
            for b in range(B):
                dst_ref[b, :, :] = a_ref[b, :, :] + r_ref[b, :, :]

        def mk(slot, h):
            return pltpu.make_async_remote_copy(
                src_ref=send_ref.at[h],
                dst_ref=recv_ref.at[slot, h],
                send_sem=send_sems.at[h],
                recv_sem=recv_sems.at[slot, h],
                device_id=(mx, my, right),
                device_id_type=pl.DeviceIdType.MESH,
            )

        def mkxy(i, qq, h, dev):
            return pltpu.make_async_remote_copy(
                src_ref=ocol_half(qq, h),
                dst_ref=ocol_half(qq, h),
                send_sem=xy_send_sems.at[i],
                recv_sem=xy_recv_sems.at[i],
                device_id=dev,
                device_id_type=pl.DeviceIdType.MESH,
            )

        partial_half(send_ref.at[0], c0, 0)
        r0 = [mk(0, 0), None]
        r0[0].start()
        partial_half(send_ref.at[1], c0, 1)
        r0[1] = mk(0, 1)
        r0[1].start()
        partial_half(phalf(0), c1, 0)
        partial_half(phalf(1), c1, 1)

        r1 = [None, None]
        for h in (0, 1):
            r0[h].wait_recv()
            r0[h].wait_send()
            add_half(send_ref.at[h], phalf(h), recv_ref.at[0, h])
            if h == 1:
                pl.semaphore_signal(
                    credit_sem, inc=1, device_id=(mx, my, left),
                    device_id_type=pl.DeviceIdType.MESH,
                )
            r1[h] = mk(1, h)
            r1[h].start()
        partial_half(phalf(0), c2, 0)
        partial_half(phalf(1), c2, 1)

        pl.semaphore_wait(credit_sem, 1)
        r2 = [None, None]
        for h in (0, 1):
            r1[h].wait_recv()
            r1[h].wait_send()
            add_half(send_ref.at[h], phalf(h), recv_ref.at[1, h])
            r2[h] = mk(0, h)
            r2[h].start()
        partial_half(phalf(0), c3, 0)
        partial_half(phalf(1), c3, 1)

        xdev = (1 - mx, my, mz)
        ydev = (mx, 1 - my, mz)
        rx = [None, None]
        ry = [None, None]
        rf = [None, None]
        for h in (0, 1):
            r2[h].wait_recv()
            add_half(ocol_half(qv, h), phalf(h), recv_ref.at[0, h])
            rx[h] = mkxy(h, qv, h, xdev)
            rx[h].start()
            ry[h] = mkxy(2 + h, qv, h, ydev)
            ry[h].start()
        for h in (0, 1):
            rx[h].wait_recv()
            rf[h] = mkxy(4 + h, qx, h, ydev)
            rf[h].start()
        for h in (0, 1):
            ry[h].wait_recv()
            rf[h].wait_recv()

        for r in rx + ry + rf + r2:
            r.wait_send()

    return pl.pallas_call(
        body,
        out_shape=jax.ShapeDtypeStruct((B, S_out, N), jnp.bfloat16),
        in_specs=[
            pl.BlockSpec(memory_space=pltpu.VMEM),
            pl.BlockSpec(memory_space=pltpu.VMEM),
        ],
        out_specs=pl.BlockSpec(memory_space=pltpu.VMEM),
        scratch_shapes=[
            pltpu.VMEM((2, B, HALF, NQ), jnp.bfloat16),
            pltpu.VMEM((2, 2, B, HALF, NQ), jnp.bfloat16),
            pltpu.VMEM((B, S_out, NQ), jnp.bfloat16),
            pltpu.SemaphoreType.DMA((2,)),
            pltpu.SemaphoreType.DMA((2, 2)),
            pltpu.SemaphoreType.DMA((6,)),
            pltpu.SemaphoreType.DMA((6,)),
            pltpu.SemaphoreType.REGULAR,
        ],
        compiler_params=pltpu.CompilerParams(
            collective_id=0, vmem_limit_bytes=60 * 1024 * 1024,
        ),
    )(o, w)


# device time: 145684 ns/iter; 1.0345x vs baseline; 1.0345x over previous
import jax
import jax.numpy as jnp
from jax import lax
from jax.experimental import pallas as pl
from jax.experimental.pallas import tpu as pltpu

N_Z = 4


def kernel(O, Wo):
    B, S, H, D = O.shape
    K = H * D
    N = Wo.shape[1]
    S_out = S // N_Z
    HALF = S_out // 2
    NQ = N // 4

    o = O.astype(jnp.bfloat16).reshape(B, S, K)
    q_idx = 2 * lax.axis_index("x") + lax.axis_index("y")
    w = lax.dynamic_slice(Wo, (0, q_idx * NQ), (K, NQ)).astype(jnp.bfloat16)

    def body(o_ref, w_ref, out_ref, send_ref, recv_ref, pbuf_ref,
             send_sems, recv_sems, xy_send_sems, xy_recv_sems, credit_sem):
        mx = lax.axis_index("x")
        my = lax.axis_index("y")
        mz = lax.axis_index("z")
        left = lax.rem(mz + N_Z - 1, N_Z)
        right = lax.rem(mz + 1, N_Z)

        qv = 2 * mx + my
        qx = 2 * (1 - mx) + my
        qy = 2 * mx + (1 - my)

        c0 = lax.rem(mz + 3, N_Z)
        c1 = lax.rem(mz + 2, N_Z)
        c2 = lax.rem(mz + 1, N_Z)
        c3 = mz

        barrier = pltpu.get_barrier_semaphore()
        for dev in ((mx, my, left), (mx, my, right),
                    (1 - mx, my, mz), (mx, 1 - my, mz)):
            pl.semaphore_signal(
                barrier, inc=1, device_id=dev,
                device_id_type=pl.DeviceIdType.MESH,
            )
        pl.semaphore_wait(barrier, 4)

        def phalf(h):
            return pbuf_ref.at[:, pl.ds(h * HALF, HALF), :]

        def ocol(qq):
            return out_ref.at[:, :, pl.ds(qq * NQ, NQ)]

        def ocol_half(qq, h):
            return out_ref.at[:, pl.ds(h * HALF, HALF), pl.ds(qq * NQ, NQ)]

        def partial_half(dst_ref, c, h):
            row0 = c * S_out + h * HALF
            for b in range(B):
                o_blk = o_ref[b, pl.ds(row0, HALF), :]
                p = jnp.dot(
                    o_blk, w_ref[:, :], preferred_element_type=jnp.float32,
                )
                dst_ref[b, :, :] = p.astype(jnp.bfloat16)

        def add_half(dst_ref, a_ref, r_ref):
            for b in range(B):
                dst_ref[b, :, :] = a_ref[b, :, :] + r_ref[b, :, :]

        def mk(slot, h):
            return pltpu.make_async_remote_copy(
                src_ref=send_ref.at[h],
                dst_ref=recv_ref.at[slot, h],
                send_sem=send_sems.at[h],
                recv_sem=recv_sems.at[slot, h],
                device_id=(mx, my, right),
                device_id_type=pl.DeviceIdType.MESH,
            )

        def mkxy(i, qq, h, dev):
            return pltpu.make_async_remote_copy(
                src_ref=ocol_half(qq, h),
                dst_ref=ocol_half(qq, h),
                send_sem=xy_send_sems.at[i],
                recv_sem=xy_recv_sems.at[i],
                device_id=dev,
                device_id_type=pl.DeviceIdType.MESH,
            )

        partial_half(send_ref.at[0], c0, 0)
        r0 = [mk(0, 0), None]
        r0[0].start()
        partial_half(send_ref.at[1], c0, 1)
        r0[1] = mk(0, 1)
        r0[1].start()
        partial_half(phalf(0), c1, 0)
        partial_half(phalf(1), c1, 1)

        r1 = [None, None]
        for h in (0, 1):
            r0[h].wait_recv()
            r0[h].wait_send()
            add_half(send_ref.at[h], phalf(h), recv_ref.at[0, h])
            if h == 1:
                pl.semaphore_signal(
                    credit_sem, inc=1, device_id=(mx, my, left),
                    device_id_type=pl.DeviceIdType.MESH,
                )
            r1[h] = mk(1, h)
            r1[h].start()
        partial_half(phalf(0), c2, 0)
        partial_half(phalf(1), c2, 1)

        pl.semaphore_wait(credit_sem, 1)
        r2 = [None, None]
        for h in (0, 1):
            r1[h].wait_recv()
            r1[h].wait_send()
            add_half(send_ref.at[h], phalf(h), recv_ref.at[1, h])
            r2[h] = mk(0, h)
            r2[h].start()
        partial_half(phalf(0), c3, 0)
        partial_half(phalf(1), c3, 1)

        xdev = (1 - mx, my, mz)
        ydev = (mx, 1 - my, mz)
        rx = [None, None]
        ry = [None, None]
        rf = [None, None]
        for h in (0, 1):
            r2[h].wait_recv()
            add_half(ocol_half(qv, h), phalf(h), recv_ref.at[0, h])
            rx[h] = mkxy(h, qv, h, xdev)
            rx[h].start()
            ry[h] = mkxy(2 + h, qv, h, ydev)
            ry[h].start()
        for h in (0, 1):
            rx[h].wait_recv()
            rf[h] = mkxy(4 + h, qx, h, ydev)
            rf[h].start()
        for h in (0, 1):
            ry[h].wait_recv()
            rf[h].wait_recv()

        for r in rx + ry + rf + r2:
            r.wait_send()

    return pl.pallas_call(
        body,
        out_shape=jax.ShapeDtypeStruct((B, S_out, N), jnp.bfloat16),
        in_specs=[
            pl.BlockSpec(memory_space=pltpu.VMEM),
            pl.BlockSpec(memory_space=pltpu.VMEM),
        ],
        out_specs=pl.BlockSpec(memory_space=pltpu.VMEM),
        scratch_shapes=[
            pltpu.VMEM((2, B, HALF, NQ), jnp.bfloat16),
            pltpu.VMEM((2, 2, B, HALF, NQ), jnp.bfloat16),
            pltpu.VMEM((B, S_out, NQ), jnp.bfloat16),
            pltpu.SemaphoreType.DMA((2,)),
            pltpu.SemaphoreType.DMA((2, 2)),
            pltpu.SemaphoreType.DMA((6,)),
            pltpu.SemaphoreType.DMA((6,)),
            pltpu.SemaphoreType.REGULAR,
        ],
        compiler_params=pltpu.CompilerParams(
            collective_id=0, vmem_limit_bytes=60 * 1024 * 1024,
        ),
    )(o, w)
